# baseline (speedup 1.0000x reference)
import functools

import jax
import jax.numpy as jnp
from jax import lax
from jax.experimental import pallas as pl
from jax.experimental.pallas import tpu as pltpu
from jax.experimental.pallas import tpu_sc as plsc

_SC_ROWS = 4096
_BB = 4096


@functools.cache
def _make_sc(V, D, B):
    info = plsc.get_sparse_core_info()
    NC, NS = info.num_cores, info.num_subcores
    NW = NC * NS
    assert B % (8 * NW) == 0
    b_per_w = B // NW
    mesh = plsc.VectorSubcoreMesh(core_axis_name="c", subcore_axis_name="s")

    @functools.partial(
        pl.kernel,
        mesh=mesh,
        out_type=jax.ShapeDtypeStruct((B, D), jnp.float32),
        scratch_types=[
            pltpu.VMEM((b_per_w,), jnp.int32),
            pltpu.VMEM((b_per_w, D), jnp.float32),
            pltpu.SemaphoreType.DMA,
        ],
    )
    def k(table_hbm, idx_hbm, out_hbm, idx_v, rows_v, sem):
        wid = lax.axis_index("s") * NC + lax.axis_index("c")
        base = wid * b_per_w
        pltpu.sync_copy(idx_hbm.at[pl.ds(base, b_per_w)], idx_v)
        pltpu.async_copy(table_hbm.at[idx_v], rows_v, sem).wait()
        pltpu.sync_copy(rows_v, out_hbm.at[pl.ds(base, b_per_w)])

    return k


def _tc_body(idx_ref, table_ref, out_ref):
    idx = idx_ref[0, 0, :]
    V = table_ref.shape[0]
    onehot = (
        idx[:, None]
        == lax.broadcasted_iota(jnp.int32, (idx.shape[0], V), 1)
    ).astype(jnp.float32)
    out_ref[...] = jax.lax.dot_general(
        onehot,
        table_ref[...],
        dimension_numbers=(((1,), (0,)), ((), ())),
        preferred_element_type=jnp.float32,
    )


@functools.cache
def _make_tc(V, D, B, BB):
    NB = B // BB

    def call(idx, table):
        idx3 = idx.reshape(NB, 1, BB)
        return pl.pallas_call(
            _tc_body,
            grid=(NB,),
            in_specs=[
                pl.BlockSpec((1, 1, BB), lambda i: (i, 0, 0)),
                pl.BlockSpec((V, D), lambda i: (0, 0)),
            ],
            out_specs=pl.BlockSpec((BB, D), lambda i: (i, 0)),
            out_shape=jax.ShapeDtypeStruct((B, D), jnp.float32),
        )(idx3, table)

    return call


@jax.jit
def kernel(inputs, table):
    idx = inputs.astype(jnp.int32)
    V, D = table.shape
    B = idx.shape[0]
    sc_out = _make_sc(V, D, _SC_ROWS)(table, idx[:_SC_ROWS])
    tc_out = _make_tc(V, D, B - _SC_ROWS, _BB)(idx[_SC_ROWS:], table)
    return jnp.concatenate([sc_out, tc_out], axis=0)


# speedup vs baseline: 1.0328x; 1.0328x over previous
import functools

import jax
import jax.numpy as jnp
from jax import lax
from jax.experimental import pallas as pl
from jax.experimental.pallas import tpu as pltpu
from jax.experimental.pallas import tpu_sc as plsc

_SC_ROWS = 8192
_BB = 4096


@functools.cache
def _make_sc(V, D, B):
    info = plsc.get_sparse_core_info()
    NC, NS = info.num_cores, info.num_subcores
    NW = NC * NS
    assert B % (8 * NW) == 0
    b_per_w = B // NW
    mesh = plsc.VectorSubcoreMesh(core_axis_name="c", subcore_axis_name="s")

    @functools.partial(
        pl.kernel,
        mesh=mesh,
        out_type=jax.ShapeDtypeStruct((B, D), jnp.float32),
        scratch_types=[
            pltpu.VMEM((b_per_w,), jnp.int32),
            pltpu.VMEM((b_per_w, D), jnp.float32),
            pltpu.SemaphoreType.DMA,
        ],
    )
    def k(table_hbm, idx_hbm, out_hbm, idx_v, rows_v, sem):
        wid = lax.axis_index("s") * NC + lax.axis_index("c")
        base = wid * b_per_w
        pltpu.sync_copy(idx_hbm.at[pl.ds(base, b_per_w)], idx_v)
        pltpu.async_copy(table_hbm.at[idx_v], rows_v, sem).wait()
        pltpu.sync_copy(rows_v, out_hbm.at[pl.ds(base, b_per_w)])

    return k


def _tc_body(idx_ref, table_ref, out_ref):
    idx = idx_ref[0, 0, :]
    V = table_ref.shape[0]
    onehot = (
        idx[:, None]
        == lax.broadcasted_iota(jnp.int32, (idx.shape[0], V), 1)
    ).astype(jnp.float32)
    out_ref[...] = jax.lax.dot_general(
        onehot,
        table_ref[...],
        dimension_numbers=(((1,), (0,)), ((), ())),
        preferred_element_type=jnp.float32,
    )


@functools.cache
def _make_tc(V, D, B, BB):
    NB = B // BB

    def call(idx, table):
        idx3 = idx.reshape(NB, 1, BB)
        return pl.pallas_call(
            _tc_body,
            grid=(NB,),
            in_specs=[
                pl.BlockSpec((1, 1, BB), lambda i: (i, 0, 0)),
                pl.BlockSpec((V, D), lambda i: (0, 0)),
            ],
            out_specs=pl.BlockSpec((BB, D), lambda i: (i, 0)),
            out_shape=jax.ShapeDtypeStruct((B, D), jnp.float32),
        )(idx3, table)

    return call


@jax.jit
def kernel(inputs, table):
    idx = inputs.astype(jnp.int32)
    V, D = table.shape
    B = idx.shape[0]
    sc_out = _make_sc(V, D, _SC_ROWS)(table, idx[:_SC_ROWS])
    tc_out = _make_tc(V, D, B - _SC_ROWS, _BB)(idx[_SC_ROWS:], table)
    return jnp.concatenate([sc_out, tc_out], axis=0)


# speedup vs baseline: 1.1528x; 1.1163x over previous
import functools

import jax
import jax.numpy as jnp
from jax import lax
from jax.experimental import pallas as pl
from jax.experimental.pallas import tpu as pltpu
from jax.experimental.pallas import tpu_sc as plsc


@functools.cache
def _make_gather(V, D, B):
    info = plsc.get_sparse_core_info()
    NC, NS = info.num_cores, info.num_subcores
    NW = NC * NS
    assert B % (8 * NW) == 0
    b_per_w = B // NW
    mesh = plsc.VectorSubcoreMesh(core_axis_name="c", subcore_axis_name="s")

    @functools.partial(
        pl.kernel,
        mesh=mesh,
        out_type=jax.ShapeDtypeStruct((B, D), jnp.float32),
        scratch_types=[
            pltpu.VMEM((b_per_w,), jnp.int32),
            pltpu.VMEM((b_per_w, D), jnp.float32),
            pltpu.SemaphoreType.DMA,
        ],
    )
    def k(table_hbm, idx_hbm, out_hbm, idx_v, rows_v, sem):
        wid = lax.axis_index("s") * NC + lax.axis_index("c")
        base = wid * b_per_w
        pltpu.sync_copy(idx_hbm.at[pl.ds(base, b_per_w)], idx_v)
        pltpu.async_copy(table_hbm.at[idx_v], rows_v, sem).wait()
        pltpu.sync_copy(rows_v, out_hbm.at[pl.ds(base, b_per_w)])

    return k


@jax.jit
def kernel(inputs, table):
    idx = inputs.astype(jnp.int32)
    return _make_gather(table.shape[0], table.shape[1], idx.shape[0])(
        table, idx
    )
